# baseline (speedup 1.0000x reference)
import functools

import jax
import jax.numpy as jnp
from jax import lax
from jax.experimental import pallas as pl
from jax.experimental.pallas import tpu as pltpu
from jax.experimental.pallas import tpu_sc as plsc

_NC = 2
_NS = 16
_NW = _NC * _NS
_K = 32
_HALF = 4
_NBUF = 2 * _HALF


@functools.cache
def _make_agg(N, D, E):
    EPW = E // _NW
    assert EPW * _NW == E
    RPT = (N // _NS) // 8 * 8
    REM = N - RPT * _NS
    ZCH = 16
    NZ = RPT // ZCH
    assert NZ * ZCH == RPT and REM % 8 == 0 and REM <= ZCH

    mesh = plsc.VectorSubcoreMesh(core_axis_name="c", subcore_axis_name="s")

    NMAIN = EPW // _K // _HALF * _HALF
    NPAIR = NMAIN // (2 * _HALF)
    assert NPAIR * 2 * _HALF == NMAIN
    TAIL = EPW - NMAIN * _K
    assert TAIL % 8 == 0 and TAIL <= _K

    @functools.partial(
        pl.kernel,
        out_type=jax.ShapeDtypeStruct((_NC, N, D), jnp.float32),
        mesh=mesh,
        scratch_types=(
            [pltpu.VMEM((_K,), jnp.int32)] * _NBUF
            + [pltpu.VMEM((_K,), jnp.int32)] * _NBUF
            + [pltpu.VMEM((_K, D), jnp.float32)] * _NBUF
            + [pltpu.VMEM((ZCH, D), jnp.float32),
               pltpu.VMEM_SHARED((N, D), jnp.float32)]
            + [pltpu.VMEM((TAIL, ), jnp.int32)] * 2
            + [pltpu.VMEM((TAIL, D), jnp.float32)]
            + [pltpu.SemaphoreType.DMA] * (3 * _NBUF + 1)
        ),
    )
    def agg(h_hbm, src_hbm, dst_hbm, out_hbm, *sc):
        sidx = sc[0:_NBUF]
        didx = sc[_NBUF:2 * _NBUF]
        rows = sc[2 * _NBUF:3 * _NBUF]
        zbuf = sc[3 * _NBUF]
        acc = sc[3 * _NBUF + 1]
        tsidx = sc[3 * _NBUF + 2]
        tdidx = sc[3 * _NBUF + 3]
        trows = sc[3 * _NBUF + 4]
        isem = sc[3 * _NBUF + 5:3 * _NBUF + 5 + _NBUF]
        gsem = sc[3 * _NBUF + 5 + _NBUF:3 * _NBUF + 5 + 2 * _NBUF]
        ssem = sc[3 * _NBUF + 5 + 2 * _NBUF:3 * _NBUF + 5 + 3 * _NBUF]
        sem = sc[-1]

        c = lax.axis_index("c")
        s = lax.axis_index("s")
        wid = s * _NC + c

        def zrow(r, carry):
            for j in range(D // 16):
                zbuf[r, pl.ds(j * 16, 16)] = jnp.zeros((16,), jnp.float32)
            return carry

        lax.fori_loop(0, ZCH, zrow, 0)
        zcopies = [
            pltpu.async_copy(zbuf, acc.at[pl.ds(s * RPT + z * ZCH, ZCH)], sem)
            for z in range(NZ)
        ]
        for cp in zcopies:
            cp.wait()

        @pl.when(s == _NS - 1)
        def _():
            pltpu.async_copy(zbuf.at[pl.ds(0, REM)],
                             acc.at[pl.ds(_NS * RPT, REM)], sem).wait()

        plsc.subcore_barrier()

        ebase = wid * EPW

        def pair(p, carry):
            for h in range(2):
                B = h * _HALF
                wbase = ebase + (2 * p + h) * _HALF * _K
                for j in range(_HALF):
                    b = B + j
                    off = wbase + j * _K

                    @pl.when(p > 0)
                    def _():
                        pltpu.make_async_copy(
                            rows[b], acc.at[didx[b]], ssem[b]).wait()

                    pltpu.async_copy(src_hbm.at[pl.ds(off, _K)], sidx[b],
                                     isem[b])
                    pltpu.async_copy(dst_hbm.at[pl.ds(off, _K)], didx[b],
                                     isem[b])
                for j in range(_HALF):
                    b = B + j
                    off = wbase + j * _K
                    pltpu.make_async_copy(
                        src_hbm.at[pl.ds(off, _K)], sidx[b], isem[b]).wait()
                    pltpu.make_async_copy(
                        dst_hbm.at[pl.ds(off, _K)], didx[b], isem[b]).wait()
                    pltpu.async_copy(h_hbm.at[sidx[b]], rows[b], gsem[b])
                for j in range(_HALF):
                    b = B + j
                    pltpu.make_async_copy(h_hbm.at[sidx[b]], rows[b],
                                          gsem[b]).wait()
                    pltpu.async_copy(rows[b], acc.at[didx[b]], ssem[b],
                                     add=True)
            return carry

        lax.fori_loop(0, NPAIR, pair, 0)
        for b in range(_NBUF):
            pltpu.make_async_copy(rows[b], acc.at[didx[b]], ssem[b]).wait()
        if TAIL:
            off = ebase + NMAIN * _K
            pltpu.sync_copy(src_hbm.at[pl.ds(off, TAIL)], tsidx)
            pltpu.sync_copy(dst_hbm.at[pl.ds(off, TAIL)], tdidx)
            pltpu.async_copy(h_hbm.at[tsidx], trows, gsem[0]).wait()
            pltpu.sync_copy(trows, acc.at[tdidx], add=True)
        plsc.subcore_barrier()
        pltpu.sync_copy(acc.at[pl.ds(s * RPT, RPT)],
                        out_hbm.at[c, pl.ds(s * RPT, RPT)])

        @pl.when(s == _NS - 1)
        def _():
            pltpu.sync_copy(acc.at[pl.ds(_NS * RPT, REM)],
                            out_hbm.at[c, pl.ds(_NS * RPT, REM)])

    return agg


def _gin_dense(h, agg, W, b, eps, g, be):
    N, D = h.shape
    H = W.shape[1]

    def body(h_ref, a_ref, w_ref, b_ref, e_ref, g_ref, be_ref, o_ref):
        hp = (1.0 + e_ref[...]) * h_ref[...] + a_ref[0] + a_ref[1]
        y = jnp.dot(hp, w_ref[...], preferred_element_type=jnp.float32)
        y = y + b_ref[...]
        mu = jnp.mean(y, axis=0, keepdims=True)
        d = y - mu
        var = jnp.mean(d * d, axis=0, keepdims=True)
        o_ref[...] = jnp.maximum(
            d * lax.rsqrt(var + 1e-5) * g_ref[...] + be_ref[...], 0.0)

    return pl.pallas_call(
        body,
        out_shape=jax.ShapeDtypeStruct((N, H), jnp.float32),
    )(h, agg, W, b.reshape(1, H), eps.reshape(1, 1), g.reshape(1, H),
      be.reshape(1, H))


def _pool_mlp(h, batch, gf, Wm0, bm0, gm0, bem0, Wm1, bm1, gm1, bem1, Wm2, bm2):
    N, Hh = h.shape
    G, GF = gf.shape
    H2 = Wm0.shape[1]
    H3 = Wm1.shape[1]

    def bn_relu(m, gv, bev):
        mu = jnp.mean(m, axis=0, keepdims=True)
        d = m - mu
        var = jnp.mean(d * d, axis=0, keepdims=True)
        return jnp.maximum(d * lax.rsqrt(var + 1e-5) * gv + bev, 0.0)

    def body(h_ref, b_ref, gf_ref, w0_ref, b0_ref, g0_ref, be0_ref,
             w1_ref, b1_ref, g1_ref, be1_ref, w2_ref, b2_ref, o_ref):
        bvec = b_ref[...]
        gids = lax.broadcasted_iota(jnp.int32, (G, N), 0)
        P = jnp.where(bvec == gids, 1.0, 0.0)
        sums = jnp.dot(P, h_ref[...], preferred_element_type=jnp.float32)
        counts = jnp.sum(P, axis=1, keepdims=True)
        pooled = sums / jnp.maximum(counts, 1.0)
        comb = jnp.concatenate([pooled, gf_ref[...]], axis=1)
        m = jnp.dot(comb, w0_ref[...], preferred_element_type=jnp.float32)
        m = bn_relu(m + b0_ref[...], g0_ref[...], be0_ref[...])
        m = jnp.dot(m, w1_ref[...], preferred_element_type=jnp.float32)
        m = bn_relu(m + b1_ref[...], g1_ref[...], be1_ref[...])
        o_ref[...] = jnp.dot(m, w2_ref[...],
                             preferred_element_type=jnp.float32) + b2_ref[...]

    return pl.pallas_call(
        body,
        out_shape=jax.ShapeDtypeStruct((G, 1), jnp.float32),
    )(h, batch.reshape(1, N), gf, Wm0, bm0.reshape(1, H2), gm0.reshape(1, H2),
      bem0.reshape(1, H2), Wm1, bm1.reshape(1, H3), gm1.reshape(1, H3),
      bem1.reshape(1, H3), Wm2, bm2.reshape(1, 1))


def kernel(x, edge_index, batch, global_feat, W0, b0, eps0, g0, be0,
           W1, b1, eps1, g1, be1, W2, b2, eps2, g2, be2,
           Wm0, bm0, gm0, bem0, Wm1, bm1, gm1, bem1, Wm2, bm2):
    N, D = x.shape
    E = edge_index.shape[1]
    G = global_feat.shape[0]
    GF = global_feat.shape[2]

    src = edge_index[0]
    dst = edge_index[1]
    agg_fn = _make_agg(N, D, E)

    h = x
    for (W, b, eps, g, be) in ((W0, b0, eps0, g0, be0),
                               (W1, b1, eps1, g1, be1),
                               (W2, b2, eps2, g2, be2)):
        a = agg_fn(h, src, dst)
        h = _gin_dense(h, a, W, b, eps, g, be)

    out = _pool_mlp(h, batch, global_feat.reshape(G, GF),
                    Wm0, bm0, gm0, bem0, Wm1, bm1, gm1, bem1, Wm2, bm2)
    return out.reshape(-1)


# speedup vs baseline: 1.1552x; 1.1552x over previous
import functools

import jax
import jax.numpy as jnp
from jax import lax
from jax.experimental import pallas as pl
from jax.experimental.pallas import tpu as pltpu
from jax.experimental.pallas import tpu_sc as plsc

_NC = 2
_NS = 16
_NW = _NC * _NS
_K = 40
_NBUF = 8


@functools.cache
def _make_agg(N, D, E):
    EPW = E // _NW
    assert EPW * _NW == E
    RPT = (N // _NS) // 8 * 8
    REM = N - RPT * _NS
    ZCH = 16
    NZ = RPT // ZCH
    assert NZ * ZCH == RPT and REM % 8 == 0 and REM <= ZCH

    mesh = plsc.VectorSubcoreMesh(core_axis_name="c", subcore_axis_name="s")

    NMAIN = EPW // _K // _NBUF * _NBUF
    NGROUP = NMAIN // _NBUF
    NTAIL = EPW // _K - NMAIN
    TAIL = EPW - (NMAIN + NTAIL) * _K
    assert TAIL % 8 == 0 and TAIL <= _K and NTAIL < _NBUF

    @functools.partial(
        pl.kernel,
        out_type=jax.ShapeDtypeStruct((_NC, N, D), jnp.float32),
        mesh=mesh,
        scratch_types=(
            [pltpu.VMEM((_K,), jnp.int32)] * _NBUF
            + [pltpu.VMEM((_K,), jnp.int32)] * _NBUF
            + [pltpu.VMEM((_K, D), jnp.float32)] * _NBUF
            + [pltpu.VMEM((ZCH, D), jnp.float32),
               pltpu.VMEM_SHARED((N, D), jnp.float32)]
            + [pltpu.VMEM((max(TAIL, 8), ), jnp.int32)] * 2
            + [pltpu.VMEM((max(TAIL, 8), D), jnp.float32)]
            + [pltpu.SemaphoreType.DMA] * (3 * _NBUF + 1)
        ),
    )
    def agg(h_hbm, src_hbm, dst_hbm, out_hbm, *sc):
        sidx = sc[0:_NBUF]
        didx = sc[_NBUF:2 * _NBUF]
        rows = sc[2 * _NBUF:3 * _NBUF]
        zbuf = sc[3 * _NBUF]
        acc = sc[3 * _NBUF + 1]
        tsidx = sc[3 * _NBUF + 2]
        tdidx = sc[3 * _NBUF + 3]
        trows = sc[3 * _NBUF + 4]
        isem = sc[3 * _NBUF + 5:3 * _NBUF + 5 + _NBUF]
        gsem = sc[3 * _NBUF + 5 + _NBUF:3 * _NBUF + 5 + 2 * _NBUF]
        ssem = sc[3 * _NBUF + 5 + 2 * _NBUF:3 * _NBUF + 5 + 3 * _NBUF]
        sem = sc[-1]

        c = lax.axis_index("c")
        s = lax.axis_index("s")
        wid = s * _NC + c

        def zrow(r, carry):
            for j in range(D // 16):
                zbuf[r, pl.ds(j * 16, 16)] = jnp.zeros((16,), jnp.float32)
            return carry

        lax.fori_loop(0, ZCH, zrow, 0)
        zcopies = [
            pltpu.async_copy(zbuf, acc.at[pl.ds(s * RPT + z * ZCH, ZCH)], sem)
            for z in range(NZ)
        ]
        for cp in zcopies:
            cp.wait()

        @pl.when(s == _NS - 1)
        def _():
            pltpu.async_copy(zbuf.at[pl.ds(0, REM)],
                             acc.at[pl.ds(_NS * RPT, REM)], sem).wait()

        plsc.subcore_barrier()

        ebase = wid * EPW

        def group(g, carry):
            for b in range(_NBUF):
                off = ebase + (g * _NBUF + b) * _K

                @pl.when(g > 0)
                def _():
                    pltpu.make_async_copy(
                        rows[b], acc.at[didx[b]], ssem[b]).wait()

                pltpu.async_copy(src_hbm.at[pl.ds(off, _K)], sidx[b], isem[b])
                pltpu.async_copy(dst_hbm.at[pl.ds(off, _K)], didx[b], isem[b])
            for b in range(_NBUF):
                off = ebase + (g * _NBUF + b) * _K
                pltpu.make_async_copy(
                    src_hbm.at[pl.ds(off, _K)], sidx[b], isem[b]).wait()
                pltpu.make_async_copy(
                    dst_hbm.at[pl.ds(off, _K)], didx[b], isem[b]).wait()
                pltpu.async_copy(h_hbm.at[sidx[b]], rows[b], gsem[b])
            for b in range(_NBUF):
                pltpu.make_async_copy(h_hbm.at[sidx[b]], rows[b],
                                      gsem[b]).wait()
                pltpu.async_copy(rows[b], acc.at[didx[b]], ssem[b], add=True)
            return carry

        lax.fori_loop(0, NGROUP, group, 0)
        for t in range(NTAIL):
            b = t
            off = ebase + (NMAIN + t) * _K
            pltpu.make_async_copy(rows[b], acc.at[didx[b]], ssem[b]).wait()
            pltpu.sync_copy(src_hbm.at[pl.ds(off, _K)], sidx[b])
            pltpu.sync_copy(dst_hbm.at[pl.ds(off, _K)], didx[b])
            pltpu.async_copy(h_hbm.at[sidx[b]], rows[b], gsem[b]).wait()
            pltpu.async_copy(rows[b], acc.at[didx[b]], ssem[b], add=True)
        for b in range(_NBUF):
            pltpu.make_async_copy(rows[b], acc.at[didx[b]], ssem[b]).wait()
        if TAIL:
            off = ebase + NMAIN * _K
            pltpu.sync_copy(src_hbm.at[pl.ds(off, TAIL)], tsidx)
            pltpu.sync_copy(dst_hbm.at[pl.ds(off, TAIL)], tdidx)
            pltpu.async_copy(h_hbm.at[tsidx], trows, gsem[0]).wait()
            pltpu.sync_copy(trows, acc.at[tdidx], add=True)
        plsc.subcore_barrier()
        pltpu.sync_copy(acc.at[pl.ds(s * RPT, RPT)],
                        out_hbm.at[c, pl.ds(s * RPT, RPT)])

        @pl.when(s == _NS - 1)
        def _():
            pltpu.sync_copy(acc.at[pl.ds(_NS * RPT, REM)],
                            out_hbm.at[c, pl.ds(_NS * RPT, REM)])

    return agg


def _gin_dense(h, agg, W, b, eps, g, be):
    N, D = h.shape
    H = W.shape[1]

    def body(h_ref, a_ref, w_ref, b_ref, e_ref, g_ref, be_ref, o_ref):
        hp = (1.0 + e_ref[...]) * h_ref[...] + a_ref[0] + a_ref[1]
        y = jnp.dot(hp, w_ref[...], preferred_element_type=jnp.float32)
        y = y + b_ref[...]
        mu = jnp.mean(y, axis=0, keepdims=True)
        d = y - mu
        var = jnp.mean(d * d, axis=0, keepdims=True)
        o_ref[...] = jnp.maximum(
            d * lax.rsqrt(var + 1e-5) * g_ref[...] + be_ref[...], 0.0)

    return pl.pallas_call(
        body,
        out_shape=jax.ShapeDtypeStruct((N, H), jnp.float32),
    )(h, agg, W, b.reshape(1, H), eps.reshape(1, 1), g.reshape(1, H),
      be.reshape(1, H))


def _pool_mlp(h, batch, gf, Wm0, bm0, gm0, bem0, Wm1, bm1, gm1, bem1, Wm2, bm2):
    N, Hh = h.shape
    G, GF = gf.shape
    H2 = Wm0.shape[1]
    H3 = Wm1.shape[1]

    def bn_relu(m, gv, bev):
        mu = jnp.mean(m, axis=0, keepdims=True)
        d = m - mu
        var = jnp.mean(d * d, axis=0, keepdims=True)
        return jnp.maximum(d * lax.rsqrt(var + 1e-5) * gv + bev, 0.0)

    def body(h_ref, b_ref, gf_ref, w0_ref, b0_ref, g0_ref, be0_ref,
             w1_ref, b1_ref, g1_ref, be1_ref, w2_ref, b2_ref, o_ref):
        bvec = b_ref[...]
        gids = lax.broadcasted_iota(jnp.int32, (G, N), 0)
        P = jnp.where(bvec == gids, 1.0, 0.0)
        sums = jnp.dot(P, h_ref[...], preferred_element_type=jnp.float32)
        counts = jnp.sum(P, axis=1, keepdims=True)
        pooled = sums / jnp.maximum(counts, 1.0)
        comb = jnp.concatenate([pooled, gf_ref[...]], axis=1)
        m = jnp.dot(comb, w0_ref[...], preferred_element_type=jnp.float32)
        m = bn_relu(m + b0_ref[...], g0_ref[...], be0_ref[...])
        m = jnp.dot(m, w1_ref[...], preferred_element_type=jnp.float32)
        m = bn_relu(m + b1_ref[...], g1_ref[...], be1_ref[...])
        o_ref[...] = jnp.dot(m, w2_ref[...],
                             preferred_element_type=jnp.float32) + b2_ref[...]

    return pl.pallas_call(
        body,
        out_shape=jax.ShapeDtypeStruct((G, 1), jnp.float32),
    )(h, batch.reshape(1, N), gf, Wm0, bm0.reshape(1, H2), gm0.reshape(1, H2),
      bem0.reshape(1, H2), Wm1, bm1.reshape(1, H3), gm1.reshape(1, H3),
      bem1.reshape(1, H3), Wm2, bm2.reshape(1, 1))


def kernel(x, edge_index, batch, global_feat, W0, b0, eps0, g0, be0,
           W1, b1, eps1, g1, be1, W2, b2, eps2, g2, be2,
           Wm0, bm0, gm0, bem0, Wm1, bm1, gm1, bem1, Wm2, bm2):
    N, D = x.shape
    E = edge_index.shape[1]
    G = global_feat.shape[0]
    GF = global_feat.shape[2]

    src = edge_index[0]
    dst = edge_index[1]
    agg_fn = _make_agg(N, D, E)

    h = x
    for (W, b, eps, g, be) in ((W0, b0, eps0, g0, be0),
                               (W1, b1, eps1, g1, be1),
                               (W2, b2, eps2, g2, be2)):
        a = agg_fn(h, src, dst)
        h = _gin_dense(h, a, W, b, eps, g, be)

    out = _pool_mlp(h, batch, global_feat.reshape(G, GF),
                    Wm0, bm0, gm0, bem0, Wm1, bm1, gm1, bem1, Wm2, bm2)
    return out.reshape(-1)


# speedup vs baseline: 1.1851x; 1.0259x over previous
import functools

import jax
import jax.numpy as jnp
from jax import lax
from jax.experimental import pallas as pl
from jax.experimental.pallas import tpu as pltpu
from jax.experimental.pallas import tpu_sc as plsc

_NC = 2
_NS = 16
_NW = _NC * _NS
_K = 40
_NBUF = 8


@functools.cache
def _make_agg(N, D, E):
    EPW = E // _NW
    assert EPW * _NW == E
    RPT = (N // _NS) // 8 * 8
    REM = N - RPT * _NS
    ZCH = 16
    NZ = RPT // ZCH
    assert NZ * ZCH == RPT and REM % 8 == 0 and REM <= ZCH

    mesh = plsc.VectorSubcoreMesh(core_axis_name="c", subcore_axis_name="s")

    NMAIN = EPW // _K // _NBUF * _NBUF
    NGROUP = NMAIN // _NBUF
    NTAIL = EPW // _K - NMAIN
    TAIL = EPW - (NMAIN + NTAIL) * _K
    assert TAIL % 8 == 0 and TAIL <= _K and NTAIL < _NBUF

    @functools.partial(
        pl.kernel,
        out_type=jax.ShapeDtypeStruct((_NC, N, D), jnp.float32),
        mesh=mesh,
        scratch_types=(
            [pltpu.VMEM((_K,), jnp.int32)] * _NBUF
            + [pltpu.VMEM((_K,), jnp.int32)] * _NBUF
            + [pltpu.VMEM((_K, D), jnp.float32)] * _NBUF
            + [pltpu.VMEM((ZCH, D), jnp.float32),
               pltpu.VMEM_SHARED((N, D), jnp.float32)]
            + [pltpu.VMEM((max(TAIL, 8), ), jnp.int32)] * 2
            + [pltpu.VMEM((max(TAIL, 8), D), jnp.float32)]
            + [pltpu.SemaphoreType.DMA] * (3 * _NBUF + 1)
        ),
    )
    def agg(h_hbm, src_hbm, dst_hbm, out_hbm, *sc):
        sidx = sc[0:_NBUF]
        didx = sc[_NBUF:2 * _NBUF]
        rows = sc[2 * _NBUF:3 * _NBUF]
        zbuf = sc[3 * _NBUF]
        acc = sc[3 * _NBUF + 1]
        tsidx = sc[3 * _NBUF + 2]
        tdidx = sc[3 * _NBUF + 3]
        trows = sc[3 * _NBUF + 4]
        isem = sc[3 * _NBUF + 5:3 * _NBUF + 5 + _NBUF]
        gsem = sc[3 * _NBUF + 5 + _NBUF:3 * _NBUF + 5 + 2 * _NBUF]
        ssem = sc[3 * _NBUF + 5 + 2 * _NBUF:3 * _NBUF + 5 + 3 * _NBUF]
        sem = sc[-1]

        c = lax.axis_index("c")
        s = lax.axis_index("s")
        wid = s * _NC + c

        ebase = wid * EPW

        for b in range(_NBUF):
            off = ebase + b * _K
            pltpu.async_copy(src_hbm.at[pl.ds(off, _K)], sidx[b], isem[b])
            pltpu.async_copy(dst_hbm.at[pl.ds(off, _K)], didx[b], isem[b])

        def zrow(r, carry):
            for j in range(D // 16):
                zbuf[r, pl.ds(j * 16, 16)] = jnp.zeros((16,), jnp.float32)
            return carry

        lax.fori_loop(0, ZCH, zrow, 0)
        zcopies = [
            pltpu.async_copy(zbuf, acc.at[pl.ds(s * RPT + z * ZCH, ZCH)], sem)
            for z in range(NZ)
        ]
        for b in range(_NBUF):
            off = ebase + b * _K
            pltpu.make_async_copy(
                src_hbm.at[pl.ds(off, _K)], sidx[b], isem[b]).wait()
            pltpu.make_async_copy(
                dst_hbm.at[pl.ds(off, _K)], didx[b], isem[b]).wait()
            pltpu.async_copy(h_hbm.at[sidx[b]], rows[b], gsem[b])
        for cp in zcopies:
            cp.wait()

        @pl.when(s == _NS - 1)
        def _():
            pltpu.async_copy(zbuf.at[pl.ds(0, REM)],
                             acc.at[pl.ds(_NS * RPT, REM)], sem).wait()

        plsc.subcore_barrier()

        def group(g, carry):
            for b in range(_NBUF):
                pltpu.make_async_copy(h_hbm.at[sidx[b]], rows[b],
                                      gsem[b]).wait()
                pltpu.async_copy(rows[b], acc.at[didx[b]], ssem[b], add=True)

            @pl.when(g < NGROUP - 1)
            def _():
                for b in range(_NBUF):
                    off = ebase + ((g + 1) * _NBUF + b) * _K
                    pltpu.make_async_copy(
                        rows[b], acc.at[didx[b]], ssem[b]).wait()
                    pltpu.async_copy(src_hbm.at[pl.ds(off, _K)], sidx[b],
                                     isem[b])
                    pltpu.async_copy(dst_hbm.at[pl.ds(off, _K)], didx[b],
                                     isem[b])
                for b in range(_NBUF):
                    off = ebase + ((g + 1) * _NBUF + b) * _K
                    pltpu.make_async_copy(
                        src_hbm.at[pl.ds(off, _K)], sidx[b], isem[b]).wait()
                    pltpu.make_async_copy(
                        dst_hbm.at[pl.ds(off, _K)], didx[b], isem[b]).wait()
                    pltpu.async_copy(h_hbm.at[sidx[b]], rows[b], gsem[b])

            return carry

        lax.fori_loop(0, NGROUP, group, 0)
        for t in range(NTAIL):
            b = t
            off = ebase + (NMAIN + t) * _K
            pltpu.make_async_copy(rows[b], acc.at[didx[b]], ssem[b]).wait()
            pltpu.sync_copy(src_hbm.at[pl.ds(off, _K)], sidx[b])
            pltpu.sync_copy(dst_hbm.at[pl.ds(off, _K)], didx[b])
            pltpu.async_copy(h_hbm.at[sidx[b]], rows[b], gsem[b]).wait()
            pltpu.async_copy(rows[b], acc.at[didx[b]], ssem[b], add=True)
        for b in range(_NBUF):
            pltpu.make_async_copy(rows[b], acc.at[didx[b]], ssem[b]).wait()
        if TAIL:
            off = ebase + NMAIN * _K
            pltpu.sync_copy(src_hbm.at[pl.ds(off, TAIL)], tsidx)
            pltpu.sync_copy(dst_hbm.at[pl.ds(off, TAIL)], tdidx)
            pltpu.async_copy(h_hbm.at[tsidx], trows, gsem[0]).wait()
            pltpu.sync_copy(trows, acc.at[tdidx], add=True)
        plsc.subcore_barrier()
        pltpu.sync_copy(acc.at[pl.ds(s * RPT, RPT)],
                        out_hbm.at[c, pl.ds(s * RPT, RPT)])

        @pl.when(s == _NS - 1)
        def _():
            pltpu.sync_copy(acc.at[pl.ds(_NS * RPT, REM)],
                            out_hbm.at[c, pl.ds(_NS * RPT, REM)])

    return agg


def _gin_dense(h, agg, W, b, eps, g, be):
    N, D = h.shape
    H = W.shape[1]

    def body(h_ref, a_ref, w_ref, b_ref, e_ref, g_ref, be_ref, o_ref):
        hp = (1.0 + e_ref[...]) * h_ref[...] + a_ref[0] + a_ref[1]
        y = jnp.dot(hp, w_ref[...], preferred_element_type=jnp.float32)
        y = y + b_ref[...]
        mu = jnp.mean(y, axis=0, keepdims=True)
        d = y - mu
        var = jnp.mean(d * d, axis=0, keepdims=True)
        o_ref[...] = jnp.maximum(
            d * lax.rsqrt(var + 1e-5) * g_ref[...] + be_ref[...], 0.0)

    return pl.pallas_call(
        body,
        out_shape=jax.ShapeDtypeStruct((N, H), jnp.float32),
    )(h, agg, W, b.reshape(1, H), eps.reshape(1, 1), g.reshape(1, H),
      be.reshape(1, H))


def _dense_pool_mlp(h, agg, W, b, eps, g, be, batch, gf,
                    Wm0, bm0, gm0, bem0, Wm1, bm1, gm1, bem1, Wm2, bm2):
    N, D = h.shape
    H = W.shape[1]
    G, GF = gf.shape
    H2 = Wm0.shape[1]
    H3 = Wm1.shape[1]

    def bn_relu(m, gv, bev):
        mu = jnp.mean(m, axis=0, keepdims=True)
        d = m - mu
        var = jnp.mean(d * d, axis=0, keepdims=True)
        return jnp.maximum(d * lax.rsqrt(var + 1e-5) * gv + bev, 0.0)

    def body(h_ref, a_ref, w_ref, b_ref, e_ref, g_ref, be_ref,
             bt_ref, gf_ref, w0_ref, b0_ref, g0_ref, be0_ref,
             w1_ref, b1_ref, g1_ref, be1_ref, w2_ref, b2_ref, o_ref):
        hp = (1.0 + e_ref[...]) * h_ref[...] + a_ref[0] + a_ref[1]
        y = jnp.dot(hp, w_ref[...], preferred_element_type=jnp.float32)
        hn = bn_relu(y + b_ref[...], g_ref[...], be_ref[...])
        bvec = bt_ref[...]
        gids = lax.broadcasted_iota(jnp.int32, (G, N), 0)
        P = jnp.where(bvec == gids, 1.0, 0.0)
        sums = jnp.dot(P, hn, preferred_element_type=jnp.float32)
        counts = jnp.sum(P, axis=1, keepdims=True)
        pooled = sums / jnp.maximum(counts, 1.0)
        comb = jnp.concatenate([pooled, gf_ref[...]], axis=1)
        m = jnp.dot(comb, w0_ref[...], preferred_element_type=jnp.float32)
        m = bn_relu(m + b0_ref[...], g0_ref[...], be0_ref[...])
        m = jnp.dot(m, w1_ref[...], preferred_element_type=jnp.float32)
        m = bn_relu(m + b1_ref[...], g1_ref[...], be1_ref[...])
        o_ref[...] = jnp.dot(m, w2_ref[...],
                             preferred_element_type=jnp.float32) + b2_ref[...]

    return pl.pallas_call(
        body,
        out_shape=jax.ShapeDtypeStruct((G, 1), jnp.float32),
    )(h, agg, W, b.reshape(1, H), eps.reshape(1, 1), g.reshape(1, H),
      be.reshape(1, H), batch.reshape(1, N), gf,
      Wm0, bm0.reshape(1, H2), gm0.reshape(1, H2), bem0.reshape(1, H2),
      Wm1, bm1.reshape(1, H3), gm1.reshape(1, H3), bem1.reshape(1, H3),
      Wm2, bm2.reshape(1, 1))


def kernel(x, edge_index, batch, global_feat, W0, b0, eps0, g0, be0,
           W1, b1, eps1, g1, be1, W2, b2, eps2, g2, be2,
           Wm0, bm0, gm0, bem0, Wm1, bm1, gm1, bem1, Wm2, bm2):
    N, D = x.shape
    E = edge_index.shape[1]
    G = global_feat.shape[0]
    GF = global_feat.shape[2]

    src = edge_index[0]
    dst = edge_index[1]
    agg_fn = _make_agg(N, D, E)

    h = x
    for (W, b, eps, g, be) in ((W0, b0, eps0, g0, be0),
                               (W1, b1, eps1, g1, be1)):
        a = agg_fn(h, src, dst)
        h = _gin_dense(h, a, W, b, eps, g, be)

    a = agg_fn(h, src, dst)
    out = _dense_pool_mlp(h, a, W2, b2, eps2, g2, be2, batch,
                          global_feat.reshape(G, GF),
                          Wm0, bm0, gm0, bem0, Wm1, bm1, gm1, bem1, Wm2, bm2)
    return out.reshape(-1)
